# baseline (speedup 1.0000x reference)
import functools

import jax
import jax.numpy as jnp
from jax.experimental import pallas as pl
from jax.experimental.pallas import tpu as pltpu

_EPS = 1e-5


def _square_distance(src, dst):
    return (jnp.sum(src ** 2, -1)[:, :, None]
            + jnp.sum(dst ** 2, -1)[:, None, :]
            - 2.0 * jnp.einsum('bnc,bmc->bnm', src, dst))


def _index_points(points, idx):
    return jax.vmap(lambda p, i: p[i])(points, idx)


def _fps_body(xyz_ref, out_ref, *, npoint, n):
    nl = n // 8
    x = xyz_ref[0, 0:8, :]
    y = xyz_ref[0, 8:16, :]
    z = xyz_ref[0, 16:24, :]
    ids = (jax.lax.broadcasted_iota(jnp.int32, (8, nl), 0) * nl
           + jax.lax.broadcasted_iota(jnp.int32, (8, nl), 1))

    def body(i, state):
        distance, farthest = state
        mask = ids == farthest
        cx = jnp.sum(jnp.where(mask, x, 0.0))
        cy = jnp.sum(jnp.where(mask, y, 0.0))
        cz = jnp.sum(jnp.where(mask, z, 0.0))
        out_ref[0, pl.ds(i, 1), :] = jnp.stack([cx, cy, cz])[None, :]
        dx = x - cx
        dy = y - cy
        dz = z - cz
        d = (dx * dx + dy * dy) + dz * dz
        distance = jnp.minimum(distance, d)
        m = jnp.max(distance)
        farthest = jnp.min(jnp.where(distance == m, ids, n))
        return distance, farthest

    distance = jnp.full((8, nl), 1e10, dtype=jnp.float32)
    jax.lax.fori_loop(0, npoint, body, (distance, jnp.int32(0)))


def _fps_pallas(xyz_t, npoint):
    B, N, _ = xyz_t.shape
    nl = N // 8
    packed = jnp.concatenate(
        [xyz_t[..., 0].reshape(B, 8, nl),
         xyz_t[..., 1].reshape(B, 8, nl),
         xyz_t[..., 2].reshape(B, 8, nl)], axis=1)
    return pl.pallas_call(
        functools.partial(_fps_body, npoint=npoint, n=N),
        out_shape=jax.ShapeDtypeStruct((B, npoint, 3), jnp.float32),
        grid=(B,),
        in_specs=[pl.BlockSpec((1, 24, nl), lambda b: (b, 0, 0))],
        out_specs=pl.BlockSpec((1, npoint, 3), lambda b: (b, 0, 0)),
        compiler_params=pltpu.CompilerParams(
            dimension_semantics=("arbitrary",)),
    )(packed)


def _ballq_body(q_ref, qn_ref, xyz_ref, xn_ref, out_ref, *, nsample, n, r2):
    q = q_ref[0]
    qn = qn_ref[0]
    data = xyz_ref[0]
    xn = xn_ref[0]
    sq = qn + xn - 2.0 * jnp.dot(q, data, preferred_element_type=jnp.float32)
    ids = jax.lax.broadcasted_iota(jnp.int32, sq.shape, 1)
    key = jnp.where(sq > r2, n, ids)
    first = None
    for k in range(nsample):
        m = jnp.min(key, axis=1, keepdims=True)
        if k == 0:
            first = jnp.where(m == n, 0, m)
            out_ref[0, :, 0:1] = first
        else:
            out_ref[0, :, k:k + 1] = jnp.where(m == n, first, m)
        key = jnp.where(key == m, n, key)


def _query_ball_pallas(radius, nsample, xyz_t, new_xyz_t):
    B, N, _ = xyz_t.shape
    S = new_xyz_t.shape[1]
    data = jnp.transpose(xyz_t, (0, 2, 1))
    xn = jnp.sum(xyz_t ** 2, -1)[:, None, :]
    qn = jnp.sum(new_xyz_t ** 2, -1)[:, :, None]
    bs = min(256, S)
    grid = (B, S // bs)
    return pl.pallas_call(
        functools.partial(_ballq_body, nsample=nsample, n=N, r2=radius ** 2),
        out_shape=jax.ShapeDtypeStruct((B, S, nsample), jnp.int32),
        grid=grid,
        in_specs=[
            pl.BlockSpec((1, bs, 3), lambda b, s: (b, s, 0)),
            pl.BlockSpec((1, bs, 1), lambda b, s: (b, s, 0)),
            pl.BlockSpec((1, 3, N), lambda b, s: (b, 0, 0)),
            pl.BlockSpec((1, 1, N), lambda b, s: (b, 0, 0)),
        ],
        out_specs=pl.BlockSpec((1, bs, nsample), lambda b, s: (b, s, 0)),
        compiler_params=pltpu.CompilerParams(
            dimension_semantics=("parallel", "arbitrary")),
    )(new_xyz_t, qn, data, xn)


def _knn_point(nsample, query, data):
    sq = _square_distance(jax.lax.stop_gradient(query),
                          jax.lax.stop_gradient(data))
    _, idx = jax.lax.top_k(-sq, nsample)
    return idx


def _conv_bn_relu(x, layer):
    if x.ndim == 4:
        x = jnp.einsum('oc,bcns->bons', layer['w'], x)
        axes = (0, 2, 3)
        shape = (1, -1, 1, 1)
    else:
        x = jnp.einsum('oc,bcn->bon', layer['w'], x)
        axes = (0, 2)
        shape = (1, -1, 1)
    mean = jnp.mean(x, axis=axes, keepdims=True)
    var = jnp.var(x, axis=axes, keepdims=True)
    x = (x - mean) / jnp.sqrt(var + _EPS)
    x = x * layer['g'].reshape(shape) + layer['b'].reshape(shape)
    return jax.nn.relu(x)


def _set_abstraction(xyz, points, npoint, radius, nsample, layers):
    xyz_t = jnp.transpose(xyz, (0, 2, 1))
    new_xyz_t = _fps_pallas(xyz_t, npoint)
    idx = _query_ball_pallas(radius, nsample, xyz_t, new_xyz_t)
    grouped_xyz = _index_points(xyz_t, idx) - new_xyz_t[:, :, None, :]
    points_t = jnp.transpose(points, (0, 2, 1))
    grouped_points = _index_points(points_t, idx)
    new_points = jnp.concatenate([grouped_xyz, grouped_points], axis=-1)
    new_points = jnp.transpose(new_points, (0, 3, 1, 2))
    for layer in layers:
        new_points = _conv_bn_relu(new_points, layer)
    new_points = jnp.max(new_points, axis=-1)
    return jnp.transpose(new_xyz_t, (0, 2, 1)), new_points


def _flow_embedding(pos1, pos2, feat1, feat2, nsample, layers):
    pos1_t = jnp.transpose(pos1, (0, 2, 1))
    pos2_t = jnp.transpose(pos2, (0, 2, 1))
    idx = _knn_point(nsample, pos1_t, pos2_t)
    pos2_grouped = _index_points(pos2_t, idx)
    pos_diff = pos2_grouped - pos1_t[:, :, None, :]
    feat2_grouped = _index_points(jnp.transpose(feat2, (0, 2, 1)), idx)
    feat1_exp = jnp.broadcast_to(
        jnp.transpose(feat1, (0, 2, 1))[:, :, None, :], feat2_grouped.shape)
    feat_new = jnp.concatenate([pos_diff, feat2_grouped, feat1_exp], axis=-1)
    feat_new = jnp.transpose(feat_new, (0, 3, 1, 2))
    for layer in layers:
        feat_new = _conv_bn_relu(feat_new, layer)
    feat_new = jnp.max(feat_new, axis=-1)
    return pos1, feat_new


def _set_upconv(pos1, pos2, feat1, feat2, nsample, mlp1, mlp2):
    pos1_t = jnp.transpose(pos1, (0, 2, 1))
    pos2_t = jnp.transpose(pos2, (0, 2, 1))
    idx = _knn_point(nsample, pos1_t, pos2_t)
    pos2_grouped = _index_points(pos2_t, idx)
    pos_diff = pos2_grouped - pos1_t[:, :, None, :]
    feat2_grouped = _index_points(jnp.transpose(feat2, (0, 2, 1)), idx)
    feat_new = jnp.concatenate([feat2_grouped, pos_diff], axis=-1)
    feat_new = jnp.transpose(feat_new, (0, 3, 1, 2))
    for layer in mlp1:
        feat_new = _conv_bn_relu(feat_new, layer)
    feat_new = jnp.max(feat_new, axis=-1)
    if feat1 is not None:
        feat_new = jnp.concatenate([feat_new, feat1], axis=1)
    for layer in mlp2:
        feat_new = _conv_bn_relu(feat_new, layer)
    return feat_new


def _feature_propagation(pos1, pos2, feat1, feat2, layers):
    pos1_t = jnp.transpose(pos1, (0, 2, 1))
    pos2_t = jnp.transpose(pos2, (0, 2, 1))
    sqrdists = _square_distance(pos1_t, pos2_t)
    neg_dists, idx = jax.lax.top_k(-sqrdists, 3)
    dists = jnp.maximum(-neg_dists, 1e-10)
    idx = jax.lax.stop_gradient(idx)
    weight = 1.0 / dists
    weight = weight / jnp.sum(weight, axis=-1, keepdims=True)
    grouped = _index_points(jnp.transpose(feat2, (0, 2, 1)), idx)
    interpolated = jnp.sum(grouped * weight[:, :, :, None], axis=2)
    interpolated = jnp.transpose(interpolated, (0, 2, 1))
    feat_new = jnp.concatenate([interpolated, feat1], axis=1)
    for layer in layers:
        feat_new = _conv_bn_relu(feat_new, layer)
    return feat_new


def _head_kernel(x_ref, w1_ref, g_ref, b_ref, w2_ref, b2_ref, out_ref):
    x = x_ref[...]
    w1 = w1_ref[...]
    y = jnp.einsum('oc,bcn->bon', w1, x,
                   preferred_element_type=jnp.float32)
    mean = jnp.mean(y, axis=(0, 2), keepdims=True)
    var = jnp.var(y, axis=(0, 2), keepdims=True)
    y = (y - mean) / jnp.sqrt(var + _EPS)
    y = y * g_ref[...][None, :, None] + b_ref[...][None, :, None]
    y = jnp.maximum(y, 0.0)
    out = jnp.einsum('oc,bcn->bon', w2_ref[...], y,
                     preferred_element_type=jnp.float32)
    out_ref[...] = out + b2_ref[...][None, :, None]


def _head(x, params):
    B, C, N = x.shape
    out = pl.pallas_call(
        _head_kernel,
        out_shape=jax.ShapeDtypeStruct((B, 3, N), jnp.float32),
    )(x, params['conv1_w'], params['bn1_g'], params['bn1_b'],
      params['conv2_w'], params['conv2_b'])
    return out


def kernel(points1, weights1, points2, weights2, params):
    r = 0.001
    pc1 = jnp.transpose(points1, (0, 2, 1))
    pc2 = jnp.transpose(points2, (0, 2, 1))
    f1 = jnp.transpose(weights1, (0, 2, 1))
    f2 = jnp.transpose(weights2, (0, 2, 1))
    l1_pc1, l1_f1 = _set_abstraction(pc1, f1, 4096, 20 * r, 16, params['sa1'])
    l2_pc1, l2_f1 = _set_abstraction(l1_pc1, l1_f1, 1024, 40 * r, 16, params['sa2'])
    l1_pc2, l1_f2 = _set_abstraction(pc2, f2, 4096, 20 * r, 16, params['sa1'])
    l2_pc2, l2_f2 = _set_abstraction(l1_pc2, l1_f2, 1024, 40 * r, 16, params['sa2'])
    return (l2_pc1, l2_f1, l2_pc2, l2_f2)


def _kernel_full(points1, weights1, points2, weights2, params):
    r = 0.001
    pc1 = jnp.transpose(points1, (0, 2, 1))
    pc2 = jnp.transpose(points2, (0, 2, 1))
    f1 = jnp.transpose(weights1, (0, 2, 1))
    f2 = jnp.transpose(weights2, (0, 2, 1))
    l1_pc1, l1_f1 = _set_abstraction(pc1, f1, 4096, 20 * r, 16, params['sa1'])
    l2_pc1, l2_f1 = _set_abstraction(l1_pc1, l1_f1, 1024, 40 * r, 16, params['sa2'])
    l1_pc2, l1_f2 = _set_abstraction(pc2, f2, 4096, 20 * r, 16, params['sa1'])
    l2_pc2, l2_f2 = _set_abstraction(l1_pc2, l1_f2, 1024, 40 * r, 16, params['sa2'])
    _, l2_f1_new = _flow_embedding(l2_pc1, l2_pc2, l2_f1, l2_f2, 64, params['fe'])
    l3_pc1, l3_f1 = _set_abstraction(l2_pc1, l2_f1_new, 256, 80 * r, 8, params['sa3'])
    l4_pc1, l4_f1 = _set_abstraction(l3_pc1, l3_f1, 64, 160 * r, 8, params['sa4'])
    l3_fnew1 = _set_upconv(l3_pc1, l4_pc1, l3_f1, l4_f1, 8,
                           params['su1_mlp1'], params['su1_mlp2'])
    l2_fnew1 = _set_upconv(l2_pc1, l3_pc1,
                           jnp.concatenate([l2_f1, l2_f1_new], axis=1),
                           l3_fnew1, 8, params['su2_mlp1'], params['su2_mlp2'])
    l1_fnew1 = _set_upconv(l1_pc1, l2_pc1, l1_f1, l2_fnew1, 8,
                           params['su3_mlp1'], params['su3_mlp2'])
    l0_fnew1 = _feature_propagation(pc1, l1_pc1, f1, l1_fnew1, params['fp'])
    out = _head(l0_fnew1, params)
    return jnp.transpose(out, (0, 2, 1))


# speedup vs baseline: 1.2319x; 1.2319x over previous
import functools

import jax
import jax.numpy as jnp
from jax.experimental import pallas as pl
from jax.experimental.pallas import tpu as pltpu

_EPS = 1e-5


def _square_distance(src, dst):
    return (jnp.sum(src ** 2, -1)[:, :, None]
            + jnp.sum(dst ** 2, -1)[:, None, :]
            - 2.0 * jnp.einsum('bnc,bmc->bnm', src, dst))


def _index_points(points, idx):
    return jax.vmap(lambda p, i: p[i])(points, idx)


def _fps_body(xyz_ref, out_ref, *, npoint, n, nb):
    nl = n // 8
    x = xyz_ref[:, 0:8, :]
    y = xyz_ref[:, 8:16, :]
    z = xyz_ref[:, 16:24, :]
    ids = (jax.lax.broadcasted_iota(jnp.int32, (nb, 8, nl), 1) * nl
           + jax.lax.broadcasted_iota(jnp.int32, (nb, 8, nl), 2))

    def body(i, state):
        distance, farthest = state
        mask = ids == farthest
        cx = jnp.sum(jnp.where(mask, x, 0.0), axis=(1, 2), keepdims=True)
        cy = jnp.sum(jnp.where(mask, y, 0.0), axis=(1, 2), keepdims=True)
        cz = jnp.sum(jnp.where(mask, z, 0.0), axis=(1, 2), keepdims=True)
        out_ref[:, pl.ds(i, 1), :] = jnp.concatenate(
            [cx[:, 0, :], cy[:, 0, :], cz[:, 0, :]], axis=1)[:, None, :]
        dx = x - cx
        dy = y - cy
        dz = z - cz
        d = (dx * dx + dy * dy) + dz * dz
        distance = jnp.minimum(distance, d)
        m = jnp.max(distance, axis=(1, 2), keepdims=True)
        farthest = jnp.min(jnp.where(distance == m, ids, n), axis=(1, 2),
                           keepdims=True)
        return distance, farthest

    distance = jnp.full((nb, 8, nl), 1e10, dtype=jnp.float32)
    farthest = jnp.zeros((nb, 1, 1), dtype=jnp.int32)
    jax.lax.fori_loop(0, npoint, body, (distance, farthest))


def _fps_pallas(xyz_t, npoint):
    B, N, _ = xyz_t.shape
    nl = N // 8
    packed = jnp.concatenate(
        [xyz_t[..., 0].reshape(B, 8, nl),
         xyz_t[..., 1].reshape(B, 8, nl),
         xyz_t[..., 2].reshape(B, 8, nl)], axis=1)
    return pl.pallas_call(
        functools.partial(_fps_body, npoint=npoint, n=N, nb=B),
        out_shape=jax.ShapeDtypeStruct((B, npoint, 3), jnp.float32),
    )(packed)


def _ballq_body(q_ref, qn_ref, xyz_ref, xn_ref, out_ref, *, nsample, n, r2):
    q = q_ref[0]
    qn = qn_ref[0]
    data = xyz_ref[0]
    xn = xn_ref[0]
    sq = qn + xn - 2.0 * jnp.dot(q, data, preferred_element_type=jnp.float32)
    ids = jax.lax.broadcasted_iota(jnp.int32, sq.shape, 1)
    key = jnp.where(sq > r2, n, ids)
    first = None
    for k in range(nsample):
        m = jnp.min(key, axis=1, keepdims=True)
        if k == 0:
            first = jnp.where(m == n, 0, m)
            out_ref[0, :, 0:1] = first
        else:
            out_ref[0, :, k:k + 1] = jnp.where(m == n, first, m)
        key = jnp.where(key == m, n, key)


def _query_ball_pallas(radius, nsample, xyz_t, new_xyz_t):
    B, N, _ = xyz_t.shape
    S = new_xyz_t.shape[1]
    data = jnp.transpose(xyz_t, (0, 2, 1))
    xn = jnp.sum(xyz_t ** 2, -1)[:, None, :]
    qn = jnp.sum(new_xyz_t ** 2, -1)[:, :, None]
    bs = min(256, S)
    grid = (B, S // bs)
    return pl.pallas_call(
        functools.partial(_ballq_body, nsample=nsample, n=N, r2=radius ** 2),
        out_shape=jax.ShapeDtypeStruct((B, S, nsample), jnp.int32),
        grid=grid,
        in_specs=[
            pl.BlockSpec((1, bs, 3), lambda b, s: (b, s, 0)),
            pl.BlockSpec((1, bs, 1), lambda b, s: (b, s, 0)),
            pl.BlockSpec((1, 3, N), lambda b, s: (b, 0, 0)),
            pl.BlockSpec((1, 1, N), lambda b, s: (b, 0, 0)),
        ],
        out_specs=pl.BlockSpec((1, bs, nsample), lambda b, s: (b, s, 0)),
        compiler_params=pltpu.CompilerParams(
            dimension_semantics=("parallel", "arbitrary")),
    )(new_xyz_t, qn, data, xn)


def _knn_point(nsample, query, data):
    sq = _square_distance(jax.lax.stop_gradient(query),
                          jax.lax.stop_gradient(data))
    _, idx = jax.lax.top_k(-sq, nsample)
    return idx


def _conv_bn_relu(x, layer):
    if x.ndim == 4:
        x = jnp.einsum('oc,bcns->bons', layer['w'], x)
        axes = (0, 2, 3)
        shape = (1, -1, 1, 1)
    else:
        x = jnp.einsum('oc,bcn->bon', layer['w'], x)
        axes = (0, 2)
        shape = (1, -1, 1)
    mean = jnp.mean(x, axis=axes, keepdims=True)
    var = jnp.var(x, axis=axes, keepdims=True)
    x = (x - mean) / jnp.sqrt(var + _EPS)
    x = x * layer['g'].reshape(shape) + layer['b'].reshape(shape)
    return jax.nn.relu(x)


def _sa_geom(xyz_t, feats_t, npoint, radius, nsample):
    new_xyz_t = _fps_pallas(xyz_t, npoint)
    idx = _query_ball_pallas(radius, nsample, xyz_t, new_xyz_t)
    grouped_xyz = _index_points(xyz_t, idx) - new_xyz_t[:, :, None, :]
    grouped_points = _index_points(feats_t, idx)
    new_points = jnp.concatenate([grouped_xyz, grouped_points], axis=-1)
    return new_xyz_t, jnp.transpose(new_points, (0, 3, 1, 2))


def _sa_mlp(new_points, layers):
    for layer in layers:
        new_points = _conv_bn_relu(new_points, layer)
    return jnp.max(new_points, axis=-1)


def _exact_onehot_dot(onehot, tab):
    hi = tab.astype(jnp.bfloat16)
    r1 = tab - hi.astype(jnp.float32)
    mid = r1.astype(jnp.bfloat16)
    lo = (r1 - mid.astype(jnp.float32)).astype(jnp.bfloat16)
    ohb = onehot.astype(jnp.bfloat16)
    return (jnp.dot(ohb, hi, preferred_element_type=jnp.float32)
            + jnp.dot(ohb, mid, preferred_element_type=jnp.float32)
            + jnp.dot(ohb, lo, preferred_element_type=jnp.float32))


def _knng_body(q_ref, qn_ref, data_ref, xn_ref, tab_ref, out_ref, *,
               k, n, pos_first):
    q = q_ref[0]
    qn = qn_ref[0]
    data = data_ref[0]
    xn = xn_ref[0]
    tab = tab_ref[0]
    sq = qn + xn - 2.0 * jnp.dot(q, data, preferred_element_type=jnp.float32)
    ids = jax.lax.broadcasted_iota(jnp.int32, sq.shape, 1)

    def body(j, sq):
        m = jnp.min(sq, axis=1, keepdims=True)
        sel = jnp.min(jnp.where(sq == m, ids, n), axis=1, keepdims=True)
        onehot = (ids == sel).astype(jnp.float32)
        row = _exact_onehot_dot(onehot, tab)
        if pos_first:
            row = jnp.concatenate([row[:, 0:3] - q, row[:, 3:]], axis=1)
        else:
            d = row.shape[1]
            row = jnp.concatenate([row[:, :d - 3], row[:, d - 3:] - q],
                                  axis=1)
        out_ref[0, pl.ds(j, 1), :, :] = row[None]
        return jnp.where(ids == sel, jnp.inf, sq)

    jax.lax.fori_loop(0, k, body, sq)


def _knn_group_pallas(k, q_pos, d_pos, feats_t, pos_first):
    B, S, _ = q_pos.shape
    N = d_pos.shape[1]
    data = jnp.transpose(d_pos, (0, 2, 1))
    xn = jnp.sum(d_pos ** 2, -1)[:, None, :]
    qn = jnp.sum(q_pos ** 2, -1)[:, :, None]
    if pos_first:
        tab = jnp.concatenate([d_pos, feats_t], axis=-1)
    else:
        tab = jnp.concatenate([feats_t, d_pos], axis=-1)
    D = tab.shape[-1]
    bs = min(256, S)
    grid = (B, S // bs)
    return pl.pallas_call(
        functools.partial(_knng_body, k=k, n=N, pos_first=pos_first),
        out_shape=jax.ShapeDtypeStruct((B, k, S, D), jnp.float32),
        grid=grid,
        in_specs=[
            pl.BlockSpec((1, bs, 3), lambda b, s: (b, s, 0)),
            pl.BlockSpec((1, bs, 1), lambda b, s: (b, s, 0)),
            pl.BlockSpec((1, 3, N), lambda b, s: (b, 0, 0)),
            pl.BlockSpec((1, 1, N), lambda b, s: (b, 0, 0)),
            pl.BlockSpec((1, N, D), lambda b, s: (b, 0, 0)),
        ],
        out_specs=pl.BlockSpec((1, k, bs, D), lambda b, s: (b, 0, s, 0)),
        compiler_params=pltpu.CompilerParams(
            dimension_semantics=("parallel", "arbitrary")),
    )(q_pos, qn, data, xn, tab)


def _fpinterp_body(q_ref, qn_ref, data_ref, xn_ref, feat_ref, out_ref, *, n):
    q = q_ref[0]
    qn = qn_ref[0]
    data = data_ref[0]
    xn = xn_ref[0]
    feat = feat_ref[0]
    sq = qn + xn - 2.0 * jnp.dot(q, data, preferred_element_type=jnp.float32)
    ids = jax.lax.broadcasted_iota(jnp.int32, sq.shape, 1)
    out = None
    wsum = jnp.zeros_like(qn)
    ws, rows = [], []
    for _ in range(3):
        m = jnp.min(sq, axis=1, keepdims=True)
        sel = jnp.min(jnp.where(sq == m, ids, n), axis=1, keepdims=True)
        w = 1.0 / jnp.maximum(m, 1e-10)
        ws.append(w)
        wsum = wsum + w
        rows.append(_exact_onehot_dot((ids == sel).astype(jnp.float32), feat))
        sq = jnp.where(ids == sel, jnp.inf, sq)
    for w, row in zip(ws, rows):
        term = (w / wsum) * row
        out = term if out is None else out + term
    out_ref[0] = out


def _fp_interp_pallas(q_pos, d_pos, feats_t):
    B, S, _ = q_pos.shape
    N, C = feats_t.shape[1], feats_t.shape[2]
    data = jnp.transpose(d_pos, (0, 2, 1))
    xn = jnp.sum(d_pos ** 2, -1)[:, None, :]
    qn = jnp.sum(q_pos ** 2, -1)[:, :, None]
    bs = min(256, S)
    grid = (B, S // bs)
    return pl.pallas_call(
        functools.partial(_fpinterp_body, n=N),
        out_shape=jax.ShapeDtypeStruct((B, S, C), jnp.float32),
        grid=grid,
        in_specs=[
            pl.BlockSpec((1, bs, 3), lambda b, s: (b, s, 0)),
            pl.BlockSpec((1, bs, 1), lambda b, s: (b, s, 0)),
            pl.BlockSpec((1, 3, N), lambda b, s: (b, 0, 0)),
            pl.BlockSpec((1, 1, N), lambda b, s: (b, 0, 0)),
            pl.BlockSpec((1, N, C), lambda b, s: (b, 0, 0)),
        ],
        out_specs=pl.BlockSpec((1, bs, C), lambda b, s: (b, s, 0)),
        compiler_params=pltpu.CompilerParams(
            dimension_semantics=("parallel", "arbitrary")),
    )(q_pos, qn, data, xn, feats_t)


def _head_kernel(x_ref, w1_ref, g_ref, b_ref, w2_ref, b2_ref, out_ref):
    x = x_ref[...]
    w1 = w1_ref[...]
    y = jnp.einsum('oc,bcn->bon', w1, x,
                   preferred_element_type=jnp.float32)
    mean = jnp.mean(y, axis=(0, 2), keepdims=True)
    var = jnp.var(y, axis=(0, 2), keepdims=True)
    y = (y - mean) / jnp.sqrt(var + _EPS)
    y = y * g_ref[...][None, :, None] + b_ref[...][None, :, None]
    y = jnp.maximum(y, 0.0)
    out = jnp.einsum('oc,bcn->bon', w2_ref[...], y,
                     preferred_element_type=jnp.float32)
    out_ref[...] = out + b2_ref[...][None, :, None]


def _head(x, params):
    B, C, N = x.shape
    out = pl.pallas_call(
        _head_kernel,
        out_shape=jax.ShapeDtypeStruct((B, 3, N), jnp.float32),
    )(x, params['conv1_w'], params['bn1_g'], params['bn1_b'],
      params['conv2_w'], params['conv2_b'])
    return out


def kernel(points1, weights1, points2, weights2, params):
    r = 0.001
    xyz0 = jnp.concatenate([points1, points2], axis=0)
    f0 = jnp.concatenate([weights1, weights2], axis=0)
    nx1, np1 = _sa_geom(xyz0, f0, 4096, 20 * r, 16)
    l1_f1 = _sa_mlp(np1[:2], params['sa1'])
    l1_f2 = _sa_mlp(np1[2:], params['sa1'])
    f1s = jnp.transpose(jnp.concatenate([l1_f1, l1_f2], axis=0), (0, 2, 1))
    nx2, np2 = _sa_geom(nx1, f1s, 1024, 40 * r, 16)
    l2_f1 = _sa_mlp(np2[:2], params['sa2'])
    l2_f2 = _sa_mlp(np2[2:], params['sa2'])
    l1_pc1 = nx1[:2]
    l2_pc1, l2_pc2 = nx2[:2], nx2[2:]

    grouped = _knn_group_pallas(
        64, l2_pc1, l2_pc2, jnp.transpose(l2_f2, (0, 2, 1)), pos_first=True)
    grouped = jnp.transpose(grouped, (0, 3, 2, 1))
    feat1_exp = jnp.broadcast_to(l2_f1[:, :, :, None],
                                 (2, 64, 1024, 64))
    feat_new = jnp.concatenate([grouped, feat1_exp], axis=1)
    for layer in params['fe']:
        feat_new = _conv_bn_relu(feat_new, layer)
    l2_f1_new = jnp.max(feat_new, axis=-1)

    nx3, np3 = _sa_geom(l2_pc1, jnp.transpose(l2_f1_new, (0, 2, 1)),
                        256, 80 * r, 8)
    l3_f1 = _sa_mlp(np3, params['sa3'])
    nx4, np4 = _sa_geom(nx3, jnp.transpose(l3_f1, (0, 2, 1)), 64, 160 * r, 8)
    l4_f1 = _sa_mlp(np4, params['sa4'])

    def upconv(q_pos, d_pos, feat1, feat2, mlp1, mlp2):
        g = _knn_group_pallas(8, q_pos, d_pos,
                              jnp.transpose(feat2, (0, 2, 1)),
                              pos_first=False)
        g = jnp.transpose(g, (0, 3, 2, 1))
        for layer in mlp1:
            g = _conv_bn_relu(g, layer)
        g = jnp.max(g, axis=-1)
        if feat1 is not None:
            g = jnp.concatenate([g, feat1], axis=1)
        for layer in mlp2:
            g = _conv_bn_relu(g, layer)
        return g

    l3_fnew1 = upconv(nx3, nx4, l3_f1, l4_f1,
                      params['su1_mlp1'], params['su1_mlp2'])
    l2_fnew1 = upconv(l2_pc1, nx3,
                      jnp.concatenate([l2_f1, l2_f1_new], axis=1),
                      l3_fnew1, params['su2_mlp1'], params['su2_mlp2'])
    l1_fnew1 = upconv(l1_pc1, l2_pc1, l1_f1, l2_fnew1,
                      params['su3_mlp1'], params['su3_mlp2'])

    interp = _fp_interp_pallas(points1, l1_pc1,
                               jnp.transpose(l1_fnew1, (0, 2, 1)))
    feat_new = jnp.concatenate(
        [jnp.transpose(interp, (0, 2, 1)),
         jnp.transpose(weights1, (0, 2, 1))], axis=1)
    for layer in params['fp']:
        feat_new = _conv_bn_relu(feat_new, layer)
    out = _head(feat_new, params)
    return jnp.transpose(out, (0, 2, 1))
